# baseline (speedup 1.0000x reference)
import functools

import jax
import jax.numpy as jnp
from jax import lax
from jax.experimental import pallas as pl
from jax.experimental.pallas import tpu as pltpu
from jax.experimental.pallas import tpu_sc as plsc

_N_E = 8192
_D = 32
_BETA = 0.25
_TOK = 8192
_TOK_TILE = 1024
_CODE_TILE = 512
_ACC_CHUNK = 2048
_N_CODE_TILES = _N_E // _CODE_TILE
_GRID = _TOK // _TOK_TILE

_NW = 32
_BPW = _TOK // _NW


def _dist_argmin_body(z_ref, zsq_ref, esq_ref, et_ref, idx_ref, loss_ref):
    z = z_ref[...].astype(jnp.bfloat16).astype(jnp.float32)
    zsq = zsq_ref[...]
    acc_v = None
    acc_e = None
    acc_i = None
    for cc in range(_N_E // _ACC_CHUNK):
        best_v = None
        best_i = None
        for c in range(_ACC_CHUNK // _CODE_TILE):
            lo = cc * _ACC_CHUNK + c * _CODE_TILE
            et = et_ref[:, lo:lo + _CODE_TILE]
            m = lax.dot_general(z, et, (((1,), (0,)), ((), ())),
                                preferred_element_type=jnp.float32)
            esq = esq_ref[0:1, lo:lo + _CODE_TILE]
            d = (zsq + esq) - 2.0 * m
            lvk = jnp.min(d, axis=1, keepdims=True)
            iota = lax.broadcasted_iota(jnp.int32, (_TOK_TILE, _CODE_TILE), 1)
            li = jnp.min(jnp.where(d == lvk, iota, _N_E), axis=1) + lo
            lv = lvk.reshape(_TOK_TILE)
            if best_v is None:
                best_v, best_i = lv, li
            else:
                upd = lv < best_v
                best_v = jnp.where(upd, lv, best_v)
                best_i = jnp.where(upd, li, best_i)
        if acc_v is None:
            acc_e, acc_i = best_v, best_i
            acc_v = best_v.astype(jnp.bfloat16).astype(jnp.float32)
        else:
            take = (best_v < acc_v) | ((best_v == acc_v) & (best_i < acc_i))
            acc_e = jnp.where(take, best_v, acc_e)
            acc_i = jnp.where(take, best_i, acc_i)
            acc_v = jnp.where(take, best_v, acc_v).astype(
                jnp.bfloat16).astype(jnp.float32)
    idx_ref[...] = acc_i.reshape(1, 1, _TOK_TILE)
    s = jnp.sum(acc_e)

    @pl.when(pl.program_id(0) == 0)
    def _init():
        loss_ref[0, 0] = s

    @pl.when(pl.program_id(0) != 0)
    def _acc():
        loss_ref[0, 0] = loss_ref[0, 0] + s


def _dist_argmin(z_flat, zsq, esq, et):
    return pl.pallas_call(
        _dist_argmin_body,
        grid=(_GRID,),
        in_specs=[
            pl.BlockSpec((_TOK_TILE, _D), lambda i: (i, 0)),
            pl.BlockSpec((_TOK_TILE, 1), lambda i: (i, 0)),
            pl.BlockSpec((1, _N_E), lambda i: (0, 0)),
            pl.BlockSpec((_D, _N_E), lambda i: (0, 0)),
        ],
        out_specs=[
            pl.BlockSpec((1, 1, _TOK_TILE), lambda i: (i, 0, 0)),
            pl.BlockSpec(memory_space=pltpu.SMEM),
        ],
        out_shape=[
            jax.ShapeDtypeStruct((_GRID, 1, _TOK_TILE), jnp.int32),
            jax.ShapeDtypeStruct((1, 1), jnp.float32),
        ],
    )(z_flat, zsq, esq, et)


def _sc_gather_body(table_hbm, idx_hbm, out_hbm, idx_v, rows_v, sem):
    wid = lax.axis_index("s") * 2 + lax.axis_index("c")
    base = wid * _BPW
    pltpu.sync_copy(idx_hbm.at[pl.ds(base, _BPW)], idx_v)
    pltpu.async_copy(table_hbm.at[idx_v], rows_v, sem).wait()
    pltpu.sync_copy(rows_v, out_hbm.at[pl.ds(base, _BPW)])


def _sc_gather(table, idx):
    mesh = plsc.VectorSubcoreMesh(core_axis_name="c", subcore_axis_name="s")
    k = pl.kernel(
        _sc_gather_body,
        out_type=jax.ShapeDtypeStruct((_TOK, _D), jnp.float32),
        mesh=mesh,
        scratch_types=[
            pltpu.VMEM((_BPW,), jnp.int32),
            pltpu.VMEM((_BPW, _D), jnp.float32),
            pltpu.SemaphoreType.DMA,
        ],
        compiler_params=pltpu.CompilerParams(use_tc_tiling_on_sc=False),
    )
    return k(table, idx)


def kernel(z, embedding_weight):
    z_p = jnp.transpose(z, (0, 2, 3, 1))
    z_flat = z_p.reshape(-1, _D)
    zsq = jnp.sum(z_flat ** 2, axis=1, keepdims=True)
    esq = jnp.sum(embedding_weight ** 2, axis=1)[None, :]
    et = embedding_weight.T

    idx3, loss_sum = _dist_argmin(z_flat, zsq, esq, et)
    idx = idx3.reshape(_TOK)
    zq_flat = _sc_gather(embedding_weight, idx)

    m = loss_sum[0, 0] * (1.0 / float(z.size))
    loss = m + _BETA * m

    zq = zq_flat.reshape(z_p.shape)
    z_q = z_p + (zq - z_p)
    z_q = jnp.transpose(z_q, (0, 3, 1, 2))
    return (z_q, loss, idx)
